# baseline (speedup 1.0000x reference)
import jax
import jax.numpy as jnp
from jax import lax
from jax.experimental import pallas as pl
from jax.experimental.pallas import tpu as pltpu
from jax.experimental.pallas import tpu_sc as plsc

ROWS = 128
COLS = 32768
KTOP = 256
LANES = 16
NUM_CORES = 2
NUM_SUBCORES = 16
NWORK = NUM_CORES * NUM_SUBCORES
SC_ROWS = 64
TC_ROWS = ROWS - SC_ROWS
TC_BLK = 8
ROWS_PER_W = SC_ROWS // NWORK
NV = COLS // LANES
BINS = 256
NGROUPS = BINS // LANES
HIST_STRIDE = BINS + 1
HIST_WORDS = LANES * HIST_STRIDE
UNROLL = 8


def _tree_add(vs):
    while len(vs) > 1:
        vs = [a + b for a, b in zip(vs[::2], vs[1::2])]
    return vs[0]


def _suffix(v):
    return lax.rev(plsc.cumsum(lax.rev(v, (0,))), (0,))


def _key(b):
    return b ^ ((b >> 31) & jnp.int32(0x7FFFFFFF))


def _topk_mask_body(xi_hbm, out_hbm, abuf, sbuf, hist, totbuf, in_sem, out_sem):
    wid = lax.axis_index("s") * NUM_CORES + lax.axis_index("c")
    lane_iota = lax.iota(jnp.int32, LANES)
    lane_off = lane_iota * HIST_STRIDE
    ones = jnp.ones((LANES,), jnp.int32)
    zeros16 = jnp.zeros((LANES,), jnp.int32)
    row0 = wid * ROWS_PER_W

    @plsc.parallel_loop(0, HIST_STRIDE, unroll=UNROLL)
    def clear0(i):
        hist[pl.ds(i * LANES, LANES)] = zeros16

    pltpu.async_copy(xi_hbm.at[row0], abuf.at[0], in_sem.at[0])

    def do_row(r, rc):
        p = r & 1
        q = 1 - p
        row = row0 + r
        pltpu.make_async_copy(xi_hbm.at[row], abuf.at[p], in_sem.at[p]).wait()

        def select_bin(k_rem):
            gv = zeros16
            for j in range(NGROUPS):
                tot = _tree_add([hist[pl.ds(l * HIST_STRIDE + j * LANES, LANES)]
                                 for l in range(LANES)])
                for l in range(LANES):
                    hist[pl.ds(l * HIST_STRIDE + j * LANES, LANES)] = zeros16
                totbuf[pl.ds(j * LANES, LANES)] = tot
                gv = jnp.where(lane_iota == j, jnp.sum(tot), gv)
            sfxg = _suffix(gv)
            geg = sfxg >= k_rem
            grp = jnp.sum(geg.astype(jnp.int32)) - 1
            above_g = jnp.sum(jnp.where(geg, 0, gv))
            tot = totbuf[pl.ds(grp * LANES, LANES)]
            sfx = _suffix(tot) + above_g
            ge = sfx >= k_rem
            selpos = jnp.sum(ge.astype(jnp.int32)) - 1
            dsel = grp * LANES + selpos
            above = above_g + jnp.sum(jnp.where(ge, 0, tot))
            cnt_sel = jnp.sum(jnp.where(lane_iota == selpos, tot, 0))
            return dsel, above, cnt_sel

        @plsc.parallel_loop(0, NV, unroll=UNROLL)
        def round1(i):
            s = _key(abuf[p, pl.ds(i * LANES, LANES)])
            digit = (s >> 24) + jnp.int32(128)
            plsc.addupdate_scatter(hist, [digit + lane_off], ones)

        dsel, above, _ = select_bin(jnp.int32(KTOP))
        prefix = dsel - jnp.int32(128)
        k_rem = jnp.int32(KTOP) - above

        @plsc.parallel_loop(0, NV, unroll=UNROLL, carry=jnp.int32(0))
        def round2(i, off):
            s = _key(abuf[p, pl.ds(i * LANES, LANES)])
            mask = (s >> 24) == prefix
            digit = (s >> 16) & jnp.int32(BINS - 1)
            plsc.addupdate_scatter(hist, [digit + lane_off], ones, mask=mask)
            plsc.store_compressed(sbuf.at[pl.ds(off, LANES)], s, mask=mask)
            cnt = plsc.all_reduce_population_count(mask)
            return off + jnp.squeeze(lax.slice(cnt, (0,), (1,)))

        n_sv = round2
        dsel, above, _ = select_bin(k_rem)
        prefix = (prefix << 8) | dsel
        k_rem = k_rem - above

        @pl.when(r >= 1)
        def _wait_prev_out():
            pltpu.make_async_copy(abuf.at[q], out_hbm.at[row - 1],
                                  out_sem.at[q]).wait()

        @pl.when(r < ROWS_PER_W - 1)
        def _prefetch_next():
            pltpu.async_copy(xi_hbm.at[row + 1], abuf.at[q], in_sem.at[q])

        nv_s = (n_sv + (LANES - 1)) // LANES
        for shift in (8, 0):
            @plsc.parallel_loop(0, nv_s, unroll=2)
            def round34(i):
                s = sbuf[pl.ds(i * LANES, LANES)]
                valid = (i * LANES + lane_iota) < n_sv
                mask = valid & ((s >> (shift + 8)) == prefix)
                digit = (s >> shift) & jnp.int32(BINS - 1)
                plsc.addupdate_scatter(hist, [digit + lane_off], ones, mask=mask)

            dsel, above, cnt_sel = select_bin(k_rem)
            prefix = (prefix << 8) | dsel
            k_rem = k_rem - above

        tvec = prefix
        quota = k_rem

        @pl.when(cnt_sel == quota)
        def _plain_mask():
            @plsc.parallel_loop(0, NV, unroll=UNROLL)
            def mask_pass(i):
                b = abuf[p, pl.ds(i * LANES, LANES)]
                keep = _key(b) >= tvec
                abuf[p, pl.ds(i * LANES, LANES)] = jnp.where(keep, b, jnp.int32(0))

        @pl.when(cnt_sel != quota)
        def _tie_mask():
            @plsc.parallel_loop(0, NV, unroll=UNROLL, carry=jnp.int32(0))
            def mask_pass(i, eq_seen):
                b = abuf[p, pl.ds(i * LANES, LANES)]
                s = _key(b)
                eq = s == tvec
                eq_rank = plsc.cumsum(eq.astype(jnp.int32)) + eq_seen
                keep = (s > tvec) | (eq & (eq_rank <= quota))
                abuf[p, pl.ds(i * LANES, LANES)] = jnp.where(keep, b, jnp.int32(0))
                cnt = plsc.all_reduce_population_count(eq)
                return eq_seen + jnp.squeeze(lax.slice(cnt, (0,), (1,)))

        pltpu.async_copy(abuf.at[p], out_hbm.at[row], out_sem.at[p])
        return rc

    lax.fori_loop(0, ROWS_PER_W, do_row, 0)
    last = ROWS_PER_W - 1
    pltpu.make_async_copy(abuf.at[last & 1], out_hbm.at[row0 + last],
                          out_sem.at[last & 1]).wait()


def _tc_body(xi_ref, out_ref):
    b = xi_ref[...]
    s = _key(b)
    ku = lax.bitcast_convert_type(s ^ jnp.int32(-2**31), jnp.uint32)
    tv = jnp.zeros((TC_BLK, 1), jnp.uint32)
    for bit in range(31, -1, -1):
        cand = tv | jnp.uint32(1 << bit)
        cnt = jnp.sum((ku >= cand).astype(jnp.int32), axis=1, keepdims=True)
        tv = jnp.where(cnt >= KTOP, cand, tv)
    out_ref[...] = jnp.where(ku >= tv, b, jnp.int32(0))


@jax.jit
def kernel(x):
    mesh = plsc.VectorSubcoreMesh(
        core_axis_name="c", subcore_axis_name="s",
        num_cores=NUM_CORES, num_subcores=NUM_SUBCORES,
    )
    run_sc = pl.kernel(
        _topk_mask_body,
        out_type=jax.ShapeDtypeStruct((SC_ROWS, COLS), jnp.int32),
        mesh=mesh,
        compiler_params=pltpu.CompilerParams(needs_layout_passes=False),
        scratch_types=[
            pltpu.VMEM((2, COLS), jnp.int32),
            pltpu.VMEM((COLS + LANES,), jnp.int32),
            pltpu.VMEM((HIST_WORDS,), jnp.int32),
            pltpu.VMEM((BINS,), jnp.int32),
            pltpu.SemaphoreType.DMA((2,)),
            pltpu.SemaphoreType.DMA((2,)),
        ],
    )
    xi = lax.bitcast_convert_type(x, jnp.int32)
    sc_out = run_sc(xi[:SC_ROWS])
    tc_out = pl.pallas_call(
        _tc_body,
        out_shape=jax.ShapeDtypeStruct((TC_ROWS, COLS), jnp.int32),
        grid=(TC_ROWS // TC_BLK,),
        in_specs=[pl.BlockSpec((TC_BLK, COLS), lambda i: (i, 0))],
        out_specs=pl.BlockSpec((TC_BLK, COLS), lambda i: (i, 0)),
    )(xi[SC_ROWS:])
    out = jnp.concatenate([sc_out, tc_out], axis=0)
    return lax.bitcast_convert_type(out, jnp.float32)


# speedup vs baseline: 1.3333x; 1.3333x over previous
import jax
import jax.numpy as jnp
from jax import lax
from jax.experimental import pallas as pl
from jax.experimental.pallas import tpu as pltpu
from jax.experimental.pallas import tpu_sc as plsc

ROWS = 128
COLS = 32768
KTOP = 256
LANES = 16
NUM_CORES = 2
NUM_SUBCORES = 16
NWORK = NUM_CORES * NUM_SUBCORES
ROWS_PER_W = ROWS // NWORK
NV = COLS // LANES
BINS = 256
NGROUPS = BINS // LANES
HIST_STRIDE = BINS + 1
HIST_WORDS = LANES * HIST_STRIDE
UNROLL = 16


def _tree_add(vs):
    while len(vs) > 1:
        vs = [a + b for a, b in zip(vs[::2], vs[1::2])]
    return vs[0]


def _suffix(v):
    return lax.rev(plsc.cumsum(lax.rev(v, (0,))), (0,))


def _key(b):
    return b ^ ((b >> 31) & jnp.int32(0x7FFFFFFF))


def _topk_mask_body(xi_hbm, out_hbm, abuf, sbuf, hist, totbuf, in_sem, out_sem):
    wid = lax.axis_index("s") * NUM_CORES + lax.axis_index("c")
    lane_iota = lax.iota(jnp.int32, LANES)
    lane_off = lane_iota * HIST_STRIDE
    ones = jnp.ones((LANES,), jnp.int32)
    zeros16 = jnp.zeros((LANES,), jnp.int32)
    row0 = wid * ROWS_PER_W

    @plsc.parallel_loop(0, HIST_STRIDE, unroll=UNROLL)
    def clear0(i):
        hist[pl.ds(i * LANES, LANES)] = zeros16

    pltpu.async_copy(xi_hbm.at[row0], abuf.at[0], in_sem.at[0])

    def do_row(r, rc):
        p = r & 1
        q = 1 - p
        row = row0 + r
        pltpu.make_async_copy(xi_hbm.at[row], abuf.at[p], in_sem.at[p]).wait()

        def select_bin(k_rem):
            gv = zeros16
            for j in range(NGROUPS):
                tot = _tree_add([hist[pl.ds(l * HIST_STRIDE + j * LANES, LANES)]
                                 for l in range(LANES)])
                for l in range(LANES):
                    hist[pl.ds(l * HIST_STRIDE + j * LANES, LANES)] = zeros16
                totbuf[pl.ds(j * LANES, LANES)] = tot
                gv = jnp.where(lane_iota == j, jnp.sum(tot), gv)
            sfxg = _suffix(gv)
            geg = sfxg >= k_rem
            grp = jnp.sum(geg.astype(jnp.int32)) - 1
            above_g = jnp.sum(jnp.where(geg, 0, gv))
            tot = totbuf[pl.ds(grp * LANES, LANES)]
            sfx = _suffix(tot) + above_g
            ge = sfx >= k_rem
            selpos = jnp.sum(ge.astype(jnp.int32)) - 1
            dsel = grp * LANES + selpos
            above = above_g + jnp.sum(jnp.where(ge, 0, tot))
            cnt_sel = jnp.sum(jnp.where(lane_iota == selpos, tot, 0))
            return dsel, above, cnt_sel

        @plsc.parallel_loop(0, NV, unroll=UNROLL)
        def round1(i):
            s = _key(abuf[p, pl.ds(i * LANES, LANES)])
            digit = (s >> 24) + jnp.int32(128)
            plsc.addupdate_scatter(hist, [digit + lane_off], ones)

        dsel, above, _ = select_bin(jnp.int32(KTOP))
        prefix = dsel - jnp.int32(128)
        k_rem = jnp.int32(KTOP) - above

        @plsc.parallel_loop(0, NV, unroll=UNROLL, carry=jnp.int32(0))
        def round2(i, off):
            s = _key(abuf[p, pl.ds(i * LANES, LANES)])
            mask = (s >> 24) == prefix
            digit = (s >> 16) & jnp.int32(BINS - 1)
            plsc.addupdate_scatter(hist, [digit + lane_off], ones, mask=mask)
            plsc.store_compressed(sbuf.at[pl.ds(off, LANES)], s, mask=mask)
            cnt = plsc.all_reduce_population_count(mask)
            return off + jnp.squeeze(lax.slice(cnt, (0,), (1,)))

        n_sv = round2
        dsel, above, _ = select_bin(k_rem)
        prefix = (prefix << 8) | dsel
        k_rem = k_rem - above

        @pl.when(r >= 1)
        def _wait_prev_out():
            pltpu.make_async_copy(abuf.at[q], out_hbm.at[row - 1],
                                  out_sem.at[q]).wait()

        @pl.when(r < ROWS_PER_W - 1)
        def _prefetch_next():
            pltpu.async_copy(xi_hbm.at[row + 1], abuf.at[q], in_sem.at[q])

        nv_s = (n_sv + (LANES - 1)) // LANES
        for shift in (8, 0):
            @plsc.parallel_loop(0, nv_s, unroll=2)
            def round34(i):
                s = sbuf[pl.ds(i * LANES, LANES)]
                valid = (i * LANES + lane_iota) < n_sv
                mask = valid & ((s >> (shift + 8)) == prefix)
                digit = (s >> shift) & jnp.int32(BINS - 1)
                plsc.addupdate_scatter(hist, [digit + lane_off], ones, mask=mask)

            dsel, above, cnt_sel = select_bin(k_rem)
            prefix = (prefix << 8) | dsel
            k_rem = k_rem - above

        tvec = prefix
        quota = k_rem

        @pl.when(cnt_sel == quota)
        def _plain_mask():
            @plsc.parallel_loop(0, NV, unroll=UNROLL)
            def mask_pass(i):
                b = abuf[p, pl.ds(i * LANES, LANES)]
                keep = _key(b) >= tvec
                abuf[p, pl.ds(i * LANES, LANES)] = jnp.where(keep, b, jnp.int32(0))

        @pl.when(cnt_sel != quota)
        def _tie_mask():
            @plsc.parallel_loop(0, NV, unroll=UNROLL, carry=jnp.int32(0))
            def mask_pass(i, eq_seen):
                b = abuf[p, pl.ds(i * LANES, LANES)]
                s = _key(b)
                eq = s == tvec
                eq_rank = plsc.cumsum(eq.astype(jnp.int32)) + eq_seen
                keep = (s > tvec) | (eq & (eq_rank <= quota))
                abuf[p, pl.ds(i * LANES, LANES)] = jnp.where(keep, b, jnp.int32(0))
                cnt = plsc.all_reduce_population_count(eq)
                return eq_seen + jnp.squeeze(lax.slice(cnt, (0,), (1,)))

        pltpu.async_copy(abuf.at[p], out_hbm.at[row], out_sem.at[p])
        return rc

    lax.fori_loop(0, ROWS_PER_W, do_row, 0)
    last = ROWS_PER_W - 1
    pltpu.make_async_copy(abuf.at[last & 1], out_hbm.at[row0 + last],
                          out_sem.at[last & 1]).wait()


@jax.jit
def kernel(x):
    mesh = plsc.VectorSubcoreMesh(
        core_axis_name="c", subcore_axis_name="s",
        num_cores=NUM_CORES, num_subcores=NUM_SUBCORES,
    )
    run = pl.kernel(
        _topk_mask_body,
        out_type=jax.ShapeDtypeStruct((ROWS, COLS), jnp.int32),
        mesh=mesh,
        compiler_params=pltpu.CompilerParams(needs_layout_passes=False),
        scratch_types=[
            pltpu.VMEM((2, COLS), jnp.int32),
            pltpu.VMEM((COLS + LANES,), jnp.int32),
            pltpu.VMEM((HIST_WORDS,), jnp.int32),
            pltpu.VMEM((BINS,), jnp.int32),
            pltpu.SemaphoreType.DMA((2,)),
            pltpu.SemaphoreType.DMA((2,)),
        ],
    )
    xi = lax.bitcast_convert_type(x, jnp.int32)
    return lax.bitcast_convert_type(run(xi), jnp.float32)


# speedup vs baseline: 1.4772x; 1.1079x over previous
import jax
import jax.numpy as jnp
from jax import lax
from jax.experimental import pallas as pl
from jax.experimental.pallas import tpu as pltpu
from jax.experimental.pallas import tpu_sc as plsc

ROWS = 128
COLS = 32768
KTOP = 256
LANES = 16
NUM_CORES = 2
NUM_SUBCORES = 16
NWORK = NUM_CORES * NUM_SUBCORES
ROWS_PER_W = ROWS // NWORK
NV = COLS // LANES
BINS = 256
NGROUPS = BINS // LANES
HIST_STRIDE = BINS + 1
HIST_WORDS = LANES * HIST_STRIDE
UNROLL = 8


def _tree_add(vs):
    while len(vs) > 1:
        vs = [a + b for a, b in zip(vs[::2], vs[1::2])]
    return vs[0]


def _suffix(v):
    return lax.rev(plsc.cumsum(lax.rev(v, (0,))), (0,))


def _key(b):
    return b ^ ((b >> 31) & jnp.int32(0x7FFFFFFF))


def _topk_mask_body(xi_hbm, out_hbm, abuf, sbuf, hist, totbuf, nref, in_sem, out_sem):
    wid = lax.axis_index("s") * NUM_CORES + lax.axis_index("c")
    lane_iota = lax.iota(jnp.int32, LANES)
    lane_off = lane_iota * HIST_STRIDE
    ones = jnp.ones((LANES,), jnp.int32)
    zeros16 = jnp.zeros((LANES,), jnp.int32)
    row0 = wid * ROWS_PER_W

    @plsc.parallel_loop(0, HIST_STRIDE, unroll=UNROLL)
    def clear0(i):
        hist[pl.ds(i * LANES, LANES)] = zeros16

    pltpu.async_copy(xi_hbm.at[row0], abuf.at[0], in_sem.at[0])

    def do_row(r, gbin):
        p = r & 1
        q = 1 - p
        row = row0 + r
        pltpu.make_async_copy(xi_hbm.at[row], abuf.at[p], in_sem.at[p]).wait()

        def select_bin(k_rem):
            gv = zeros16
            for j in range(NGROUPS):
                tot = _tree_add([hist[pl.ds(l * HIST_STRIDE + j * LANES, LANES)]
                                 for l in range(LANES)])
                for l in range(LANES):
                    hist[pl.ds(l * HIST_STRIDE + j * LANES, LANES)] = zeros16
                totbuf[pl.ds(j * LANES, LANES)] = tot
                gv = jnp.where(lane_iota == j, jnp.sum(tot), gv)
            sfxg = _suffix(gv)
            geg = sfxg >= k_rem
            grp = jnp.sum(geg.astype(jnp.int32)) - 1
            above_g = jnp.sum(jnp.where(geg, 0, gv))
            tot = totbuf[pl.ds(grp * LANES, LANES)]
            sfx = _suffix(tot) + above_g
            ge = sfx >= k_rem
            selpos = jnp.sum(ge.astype(jnp.int32)) - 1
            dsel = grp * LANES + selpos
            above = above_g + jnp.sum(jnp.where(ge, 0, tot))
            cnt_sel = jnp.sum(jnp.where(lane_iota == selpos, tot, 0))
            return dsel, above, cnt_sel

        @plsc.parallel_loop(0, NV, unroll=UNROLL, carry=jnp.int32(0))
        def pass_a(i, off):
            s = _key(abuf[p, pl.ds(i * LANES, LANES)])
            digit = (s >> 24) + jnp.int32(128)
            plsc.addupdate_scatter(hist, [digit + lane_off], ones)
            mask = digit >= gbin
            plsc.store_compressed(sbuf.at[pl.ds(off, LANES)], s, mask=mask)
            cnt = plsc.all_reduce_population_count(mask)
            return off + jnp.squeeze(lax.slice(cnt, (0,), (1,)))

        n_c = pass_a
        dsel, above, _ = select_bin(jnp.int32(KTOP))
        dbin = dsel
        prefix = dsel - jnp.int32(128)
        k_rem = jnp.int32(KTOP) - above

        @pl.when(dbin >= gbin)
        def _r2_fast():
            nv_c = (n_c + (LANES - 1)) // LANES

            @plsc.parallel_loop(0, nv_c, unroll=2)
            def r2f(i):
                s = sbuf[pl.ds(i * LANES, LANES)]
                valid = (i * LANES + lane_iota) < n_c
                mask = valid & ((s >> 24) == prefix)
                digit = (s >> 16) & jnp.int32(BINS - 1)
                plsc.addupdate_scatter(hist, [digit + lane_off], ones, mask=mask)

            nref[0] = n_c

        @pl.when(dbin < gbin)
        def _r2_full():
            @plsc.parallel_loop(0, NV, unroll=UNROLL, carry=jnp.int32(0))
            def r2s(i, off):
                s = _key(abuf[p, pl.ds(i * LANES, LANES)])
                mask = (s >> 24) == prefix
                digit = (s >> 16) & jnp.int32(BINS - 1)
                plsc.addupdate_scatter(hist, [digit + lane_off], ones, mask=mask)
                plsc.store_compressed(sbuf.at[pl.ds(off, LANES)], s, mask=mask)
                cnt = plsc.all_reduce_population_count(mask)
                return off + jnp.squeeze(lax.slice(cnt, (0,), (1,)))

            nref[0] = r2s

        n_sb = nref[0]
        dsel, above, _ = select_bin(k_rem)
        prefix = (prefix << 8) | dsel
        k_rem = k_rem - above

        @pl.when(r >= 1)
        def _wait_prev_out():
            pltpu.make_async_copy(abuf.at[q], out_hbm.at[row - 1],
                                  out_sem.at[q]).wait()

        @pl.when(r < ROWS_PER_W - 1)
        def _prefetch_next():
            pltpu.async_copy(xi_hbm.at[row + 1], abuf.at[q], in_sem.at[q])

        nv_s = (n_sb + (LANES - 1)) // LANES
        for shift in (8, 0):
            @plsc.parallel_loop(0, nv_s, unroll=2)
            def round34(i):
                s = sbuf[pl.ds(i * LANES, LANES)]
                valid = (i * LANES + lane_iota) < n_sb
                mask = valid & ((s >> (shift + 8)) == prefix)
                digit = (s >> shift) & jnp.int32(BINS - 1)
                plsc.addupdate_scatter(hist, [digit + lane_off], ones, mask=mask)

            dsel, above, cnt_sel = select_bin(k_rem)
            prefix = (prefix << 8) | dsel
            k_rem = k_rem - above

        tvec = prefix
        quota = k_rem

        @pl.when(cnt_sel == quota)
        def _plain_mask():
            @plsc.parallel_loop(0, NV, unroll=UNROLL)
            def mask_pass(i):
                b = abuf[p, pl.ds(i * LANES, LANES)]
                keep = _key(b) >= tvec
                abuf[p, pl.ds(i * LANES, LANES)] = jnp.where(keep, b, jnp.int32(0))

        @pl.when(cnt_sel != quota)
        def _tie_mask():
            @plsc.parallel_loop(0, NV, unroll=UNROLL, carry=jnp.int32(0))
            def mask_pass(i, eq_seen):
                b = abuf[p, pl.ds(i * LANES, LANES)]
                s = _key(b)
                eq = s == tvec
                eq_rank = plsc.cumsum(eq.astype(jnp.int32)) + eq_seen
                keep = (s > tvec) | (eq & (eq_rank <= quota))
                abuf[p, pl.ds(i * LANES, LANES)] = jnp.where(keep, b, jnp.int32(0))
                cnt = plsc.all_reduce_population_count(eq)
                return eq_seen + jnp.squeeze(lax.slice(cnt, (0,), (1,)))

        pltpu.async_copy(abuf.at[p], out_hbm.at[row], out_sem.at[p])
        return dbin

    lax.fori_loop(0, ROWS_PER_W, do_row, jnp.int32(BINS))
    last = ROWS_PER_W - 1
    pltpu.make_async_copy(abuf.at[last & 1], out_hbm.at[row0 + last],
                          out_sem.at[last & 1]).wait()


@jax.jit
def kernel(x):
    mesh = plsc.VectorSubcoreMesh(
        core_axis_name="c", subcore_axis_name="s",
        num_cores=NUM_CORES, num_subcores=NUM_SUBCORES,
    )
    run = pl.kernel(
        _topk_mask_body,
        out_type=jax.ShapeDtypeStruct((ROWS, COLS), jnp.int32),
        mesh=mesh,
        compiler_params=pltpu.CompilerParams(needs_layout_passes=False),
        scratch_types=[
            pltpu.VMEM((2, COLS), jnp.int32),
            pltpu.VMEM((COLS + LANES,), jnp.int32),
            pltpu.VMEM((HIST_WORDS,), jnp.int32),
            pltpu.VMEM((BINS,), jnp.int32),
            pltpu.SMEM((1,), jnp.int32),
            pltpu.SemaphoreType.DMA((2,)),
            pltpu.SemaphoreType.DMA((2,)),
        ],
    )
    xi = lax.bitcast_convert_type(x, jnp.int32)
    return lax.bitcast_convert_type(run(xi), jnp.float32)


# speedup vs baseline: 1.5598x; 1.0559x over previous
import jax
import jax.numpy as jnp
from jax import lax
from jax.experimental import pallas as pl
from jax.experimental.pallas import tpu as pltpu
from jax.experimental.pallas import tpu_sc as plsc

ROWS = 128
COLS = 32768
KTOP = 256
LANES = 16
NUM_CORES = 2
NUM_SUBCORES = 16
NWORK = NUM_CORES * NUM_SUBCORES
ROWS_PER_W = ROWS // NWORK
NV = COLS // LANES
BINS = 256
NGROUPS = BINS // LANES
HIST_STRIDE = BINS + 1
HIST_WORDS = LANES * HIST_STRIDE
UNROLL = 8


def _tree_add(vs):
    while len(vs) > 1:
        vs = [a + b for a, b in zip(vs[::2], vs[1::2])]
    return vs[0]


def _suffix(v):
    return lax.rev(plsc.cumsum(lax.rev(v, (0,))), (0,))


def _key(b):
    return b ^ ((b >> 31) & jnp.int32(0x7FFFFFFF))


def _topk_mask_body(xi_hbm, out_hbm, abuf, sbuf, hist, totbuf, nref, in_sem, out_sem):
    wid = lax.axis_index("s") * NUM_CORES + lax.axis_index("c")
    lane_iota = lax.iota(jnp.int32, LANES)
    lane_off = lane_iota * HIST_STRIDE
    ones = jnp.ones((LANES,), jnp.int32)
    zeros16 = jnp.zeros((LANES,), jnp.int32)
    row0 = wid * ROWS_PER_W

    @plsc.parallel_loop(0, HIST_STRIDE, unroll=UNROLL)
    def clear0(i):
        hist[pl.ds(i * LANES, LANES)] = zeros16

    pltpu.async_copy(xi_hbm.at[row0], abuf.at[0], in_sem.at[0])

    def do_row(r, gbin):
        p = r & 1
        q = 1 - p
        row = row0 + r
        pltpu.make_async_copy(xi_hbm.at[row], abuf.at[p], in_sem.at[p]).wait()

        def select_bin(k_rem):
            gv = zeros16
            for j in range(NGROUPS):
                tot = _tree_add([hist[pl.ds(l * HIST_STRIDE + j * LANES, LANES)]
                                 for l in range(LANES)])
                for l in range(LANES):
                    hist[pl.ds(l * HIST_STRIDE + j * LANES, LANES)] = zeros16
                totbuf[pl.ds(j * LANES, LANES)] = tot
                gv = jnp.where(lane_iota == j, jnp.sum(tot), gv)
            sfxg = _suffix(gv)
            geg = sfxg >= k_rem
            grp = jnp.sum(geg.astype(jnp.int32)) - 1
            above_g = jnp.sum(jnp.where(geg, 0, gv))
            tot = totbuf[pl.ds(grp * LANES, LANES)]
            sfx = _suffix(tot) + above_g
            ge = sfx >= k_rem
            selpos = jnp.sum(ge.astype(jnp.int32)) - 1
            dsel = grp * LANES + selpos
            above = above_g + jnp.sum(jnp.where(ge, 0, tot))
            cnt_sel = jnp.sum(jnp.where(lane_iota == selpos, tot, 0))
            return dsel, above, cnt_sel

        gbin0 = gbin - jnp.int32(128)
        lane_off128 = lane_off + jnp.int32(128)

        @plsc.parallel_loop(0, NV, unroll=UNROLL, carry=jnp.int32(0))
        def pass_a(i, off):
            s = _key(abuf[p, pl.ds(i * LANES, LANES)])
            digit0 = s >> 24
            plsc.addupdate_scatter(hist, [digit0 + lane_off128], ones)
            mask = digit0 >= gbin0
            plsc.store_compressed(sbuf.at[pl.ds(off, LANES)], s, mask=mask)
            cnt = plsc.all_reduce_population_count(mask)
            return off + jnp.squeeze(lax.slice(cnt, (0,), (1,)))

        n_c = pass_a
        dsel, above, _ = select_bin(jnp.int32(KTOP))
        dbin = dsel
        prefix = dsel - jnp.int32(128)
        k_rem = jnp.int32(KTOP) - above

        @pl.when(dbin >= gbin)
        def _r2_fast():
            nv_c = (n_c + (LANES - 1)) // LANES

            @plsc.parallel_loop(0, nv_c, unroll=2)
            def r2f(i):
                s = sbuf[pl.ds(i * LANES, LANES)]
                valid = (i * LANES + lane_iota) < n_c
                mask = valid & ((s >> 24) == prefix)
                digit = (s >> 16) & jnp.int32(BINS - 1)
                plsc.addupdate_scatter(hist, [digit + lane_off], ones, mask=mask)

            nref[0] = n_c

        @pl.when(dbin < gbin)
        def _r2_full():
            @plsc.parallel_loop(0, NV, unroll=UNROLL, carry=jnp.int32(0))
            def r2s(i, off):
                s = _key(abuf[p, pl.ds(i * LANES, LANES)])
                mask = (s >> 24) == prefix
                digit = (s >> 16) & jnp.int32(BINS - 1)
                plsc.addupdate_scatter(hist, [digit + lane_off], ones, mask=mask)
                plsc.store_compressed(sbuf.at[pl.ds(off, LANES)], s, mask=mask)
                cnt = plsc.all_reduce_population_count(mask)
                return off + jnp.squeeze(lax.slice(cnt, (0,), (1,)))

            nref[0] = r2s

        n_sb = nref[0]
        dsel, above, _ = select_bin(k_rem)
        prefix = (prefix << 8) | dsel
        k_rem = k_rem - above

        @pl.when(r >= 1)
        def _wait_prev_out():
            pltpu.make_async_copy(abuf.at[q], out_hbm.at[row - 1],
                                  out_sem.at[q]).wait()

        @pl.when(r < ROWS_PER_W - 1)
        def _prefetch_next():
            pltpu.async_copy(xi_hbm.at[row + 1], abuf.at[q], in_sem.at[q])

        nv_s = (n_sb + (LANES - 1)) // LANES
        for shift in (8, 0):
            @plsc.parallel_loop(0, nv_s, unroll=2)
            def round34(i):
                s = sbuf[pl.ds(i * LANES, LANES)]
                valid = (i * LANES + lane_iota) < n_sb
                mask = valid & ((s >> (shift + 8)) == prefix)
                digit = (s >> shift) & jnp.int32(BINS - 1)
                plsc.addupdate_scatter(hist, [digit + lane_off], ones, mask=mask)

            dsel, above, cnt_sel = select_bin(k_rem)
            prefix = (prefix << 8) | dsel
            k_rem = k_rem - above

        tvec = prefix
        quota = k_rem

        @pl.when(cnt_sel == quota)
        def _plain_mask():
            @plsc.parallel_loop(0, NV, unroll=UNROLL)
            def mask_pass(i):
                b = abuf[p, pl.ds(i * LANES, LANES)]
                keep = _key(b) >= tvec
                abuf[p, pl.ds(i * LANES, LANES)] = jnp.where(keep, b, jnp.int32(0))

        @pl.when(cnt_sel != quota)
        def _tie_mask():
            @plsc.parallel_loop(0, NV, unroll=UNROLL, carry=jnp.int32(0))
            def mask_pass(i, eq_seen):
                b = abuf[p, pl.ds(i * LANES, LANES)]
                s = _key(b)
                eq = s == tvec
                eq_rank = plsc.cumsum(eq.astype(jnp.int32)) + eq_seen
                keep = (s > tvec) | (eq & (eq_rank <= quota))
                abuf[p, pl.ds(i * LANES, LANES)] = jnp.where(keep, b, jnp.int32(0))
                cnt = plsc.all_reduce_population_count(eq)
                return eq_seen + jnp.squeeze(lax.slice(cnt, (0,), (1,)))

        pltpu.async_copy(abuf.at[p], out_hbm.at[row], out_sem.at[p])
        return dbin

    lax.fori_loop(0, ROWS_PER_W, do_row, jnp.int32(192))
    last = ROWS_PER_W - 1
    pltpu.make_async_copy(abuf.at[last & 1], out_hbm.at[row0 + last],
                          out_sem.at[last & 1]).wait()


@jax.jit
def kernel(x):
    mesh = plsc.VectorSubcoreMesh(
        core_axis_name="c", subcore_axis_name="s",
        num_cores=NUM_CORES, num_subcores=NUM_SUBCORES,
    )
    run = pl.kernel(
        _topk_mask_body,
        out_type=jax.ShapeDtypeStruct((ROWS, COLS), jnp.int32),
        mesh=mesh,
        compiler_params=pltpu.CompilerParams(needs_layout_passes=False),
        scratch_types=[
            pltpu.VMEM((2, COLS), jnp.int32),
            pltpu.VMEM((COLS + LANES,), jnp.int32),
            pltpu.VMEM((HIST_WORDS,), jnp.int32),
            pltpu.VMEM((BINS,), jnp.int32),
            pltpu.SMEM((1,), jnp.int32),
            pltpu.SemaphoreType.DMA((2,)),
            pltpu.SemaphoreType.DMA((2,)),
        ],
    )
    xi = lax.bitcast_convert_type(x, jnp.int32)
    return lax.bitcast_convert_type(run(xi), jnp.float32)


# speedup vs baseline: 1.5611x; 1.0009x over previous
import jax
import jax.numpy as jnp
from jax import lax
from jax.experimental import pallas as pl
from jax.experimental.pallas import tpu as pltpu
from jax.experimental.pallas import tpu_sc as plsc

ROWS = 128
COLS = 32768
KTOP = 256
LANES = 16
NUM_CORES = 2
NUM_SUBCORES = 16
NWORK = NUM_CORES * NUM_SUBCORES
ROWS_PER_W = ROWS // NWORK
NV = COLS // LANES
BINS = 256
NGROUPS = BINS // LANES
HIST_STRIDE = BINS + 1
HIST_WORDS = LANES * HIST_STRIDE
UNROLL = 8


def _tree_add(vs):
    while len(vs) > 1:
        vs = [a + b for a, b in zip(vs[::2], vs[1::2])]
    return vs[0]


def _suffix(v):
    return lax.rev(plsc.cumsum(lax.rev(v, (0,))), (0,))


def _key(b):
    return b ^ ((b >> 31) & jnp.int32(0x7FFFFFFF))


def _topk_mask_body(xi_hbm, out_hbm, abuf, sbuf, hist, totbuf, nref, in_sem, out_sem):
    wid = lax.axis_index("s") * NUM_CORES + lax.axis_index("c")
    lane_iota = lax.iota(jnp.int32, LANES)
    lane_off = lane_iota * HIST_STRIDE
    ones = jnp.ones((LANES,), jnp.int32)
    zeros16 = jnp.zeros((LANES,), jnp.int32)
    row0 = wid * ROWS_PER_W

    @plsc.parallel_loop(0, HIST_STRIDE, unroll=UNROLL)
    def clear0(i):
        hist[pl.ds(i * LANES, LANES)] = zeros16

    pltpu.async_copy(xi_hbm.at[row0], abuf.at[0], in_sem.at[0])

    def do_row(r, gbin):
        p = r & 1
        q = 1 - p
        row = row0 + r
        pltpu.make_async_copy(xi_hbm.at[row], abuf.at[p], in_sem.at[p]).wait()

        def select_bin(k_rem):
            gv = zeros16
            for j in range(NGROUPS):
                tot = _tree_add([hist[pl.ds(l * HIST_STRIDE + j * LANES, LANES)]
                                 for l in range(LANES)])
                for l in range(LANES):
                    hist[pl.ds(l * HIST_STRIDE + j * LANES, LANES)] = zeros16
                totbuf[pl.ds(j * LANES, LANES)] = tot
                gv = jnp.where(lane_iota == j, jnp.sum(tot), gv)
            sfxg = _suffix(gv)
            geg = sfxg >= k_rem
            grp = jnp.sum(geg.astype(jnp.int32)) - 1
            above_g = jnp.sum(jnp.where(geg, 0, gv))
            tot = totbuf[pl.ds(grp * LANES, LANES)]
            sfx = _suffix(tot) + above_g
            ge = sfx >= k_rem
            selpos = jnp.sum(ge.astype(jnp.int32)) - 1
            dsel = grp * LANES + selpos
            above = above_g + jnp.sum(jnp.where(ge, 0, tot))
            cnt_sel = jnp.sum(jnp.where(lane_iota == selpos, tot, 0))
            return dsel, above, cnt_sel

        gbin0 = gbin - jnp.int32(128)
        lane_off128 = lane_off + jnp.int32(128)

        @plsc.parallel_loop(0, NV, unroll=UNROLL, carry=jnp.int32(0))
        def pass_a(i, off):
            s = _key(abuf[p, pl.ds(i * LANES, LANES)])
            digit0 = s >> 24
            plsc.addupdate_scatter(hist, [digit0 + lane_off128], ones)
            mask = digit0 >= gbin0
            plsc.store_compressed(sbuf.at[pl.ds(off, LANES)], s, mask=mask)
            cnt = plsc.all_reduce_population_count(mask)
            return off + jnp.squeeze(lax.slice(cnt, (0,), (1,)))

        n_c = pass_a
        dsel, above, _ = select_bin(jnp.int32(KTOP))
        dbin = dsel
        prefix = dsel - jnp.int32(128)
        k_rem = jnp.int32(KTOP) - above

        @pl.when(dbin >= gbin)
        def _r2_fast():
            nv_c = (n_c + (LANES - 1)) // LANES

            @plsc.parallel_loop(0, nv_c, unroll=2)
            def r2f(i):
                s = sbuf[pl.ds(i * LANES, LANES)]
                valid = (i * LANES + lane_iota) < n_c
                mask = valid & ((s >> 24) == prefix)
                digit = (s >> 16) & jnp.int32(BINS - 1)
                plsc.addupdate_scatter(hist, [digit + lane_off], ones, mask=mask)

            nref[0] = n_c

        @pl.when(dbin < gbin)
        def _r2_full():
            @plsc.parallel_loop(0, NV, unroll=UNROLL, carry=jnp.int32(0))
            def r2s(i, off):
                s = _key(abuf[p, pl.ds(i * LANES, LANES)])
                mask = (s >> 24) == prefix
                digit = (s >> 16) & jnp.int32(BINS - 1)
                plsc.addupdate_scatter(hist, [digit + lane_off], ones, mask=mask)
                plsc.store_compressed(sbuf.at[pl.ds(off, LANES)], s, mask=mask)
                cnt = plsc.all_reduce_population_count(mask)
                return off + jnp.squeeze(lax.slice(cnt, (0,), (1,)))

            nref[0] = r2s

        n_sb = nref[0]
        dsel, above, _ = select_bin(k_rem)
        prefix = (prefix << 8) | dsel
        k_rem = k_rem - above

        @pl.when(r >= 1)
        def _wait_prev_out():
            pltpu.make_async_copy(abuf.at[q], out_hbm.at[row - 1],
                                  out_sem.at[q]).wait()

        @pl.when(r < ROWS_PER_W - 1)
        def _prefetch_next():
            pltpu.async_copy(xi_hbm.at[row + 1], abuf.at[q], in_sem.at[q])

        nv_s = (n_sb + (LANES - 1)) // LANES
        for shift in (8, 0):
            @plsc.parallel_loop(0, nv_s, unroll=2)
            def round34(i):
                s = sbuf[pl.ds(i * LANES, LANES)]
                valid = (i * LANES + lane_iota) < n_sb
                mask = valid & ((s >> (shift + 8)) == prefix)
                digit = (s >> shift) & jnp.int32(BINS - 1)
                plsc.addupdate_scatter(hist, [digit + lane_off], ones, mask=mask)

            dsel, above, cnt_sel = select_bin(k_rem)
            prefix = (prefix << 8) | dsel
            k_rem = k_rem - above

        tvec = prefix
        quota = k_rem

        @pl.when(cnt_sel == quota)
        def _plain_mask():
            @plsc.parallel_loop(0, NV, unroll=UNROLL)
            def mask_pass(i):
                b = abuf[p, pl.ds(i * LANES, LANES)]
                keep = _key(b) >= tvec
                abuf[p, pl.ds(i * LANES, LANES)] = jnp.where(keep, b, jnp.int32(0))

        @pl.when(cnt_sel != quota)
        def _tie_mask():
            @plsc.parallel_loop(0, NV, unroll=UNROLL, carry=jnp.int32(0))
            def mask_pass(i, eq_seen):
                b = abuf[p, pl.ds(i * LANES, LANES)]
                s = _key(b)
                eq = s == tvec
                eq_rank = plsc.cumsum(eq.astype(jnp.int32)) + eq_seen
                keep = (s > tvec) | (eq & (eq_rank <= quota))
                abuf[p, pl.ds(i * LANES, LANES)] = jnp.where(keep, b, jnp.int32(0))
                cnt = plsc.all_reduce_population_count(eq)
                return eq_seen + jnp.squeeze(lax.slice(cnt, (0,), (1,)))

        pltpu.async_copy(abuf.at[p], out_hbm.at[row], out_sem.at[p])
        return dbin

    lax.fori_loop(0, ROWS_PER_W, do_row, jnp.int32(192))
    last = ROWS_PER_W - 1
    pltpu.make_async_copy(abuf.at[last & 1], out_hbm.at[row0 + last],
                          out_sem.at[last & 1]).wait()


@jax.jit
def kernel(x):
    mesh = plsc.VectorSubcoreMesh(
        core_axis_name="c", subcore_axis_name="s",
        num_cores=NUM_CORES, num_subcores=NUM_SUBCORES,
    )
    run = pl.kernel(
        _topk_mask_body,
        out_type=jax.ShapeDtypeStruct((ROWS, COLS), jnp.int32),
        mesh=mesh,
        compiler_params=pltpu.CompilerParams(
            needs_layout_passes=False,
            disable_bounds_checks=True,
            skip_device_barrier=True,
        ),
        scratch_types=[
            pltpu.VMEM((2, COLS), jnp.int32),
            pltpu.VMEM((COLS + LANES,), jnp.int32),
            pltpu.VMEM((HIST_WORDS,), jnp.int32),
            pltpu.VMEM((BINS,), jnp.int32),
            pltpu.SMEM((1,), jnp.int32),
            pltpu.SemaphoreType.DMA((2,)),
            pltpu.SemaphoreType.DMA((2,)),
        ],
    )
    xi = lax.bitcast_convert_type(x, jnp.int32)
    return lax.bitcast_convert_type(run(xi), jnp.float32)
